# baseline (speedup 1.0000x reference)
import functools

import jax
import jax.numpy as jnp
from jax import lax
from jax.experimental import pallas as pl
from jax.experimental.pallas import tpu as pltpu
from jax.experimental.pallas import tpu_sc as plsc

DIM = 64
LANES = 16
NUM_CORES = 2
NUM_SUBCORES = 16
NUM_WORKERS = NUM_CORES * NUM_SUBCORES
CHUNK = 128


def _scores_sc(users, items, user_table, item_table):
    batch = users.shape[0]
    b_per_w = batch // NUM_WORKERS
    n_chunk = b_per_w // CHUNK
    groups_per_chunk = CHUNK // LANES

    mesh = plsc.VectorSubcoreMesh(core_axis_name="c", subcore_axis_name="s")

    @functools.partial(
        pl.kernel,
        mesh=mesh,
        out_type=jax.ShapeDtypeStruct((batch,), jnp.float32),
        scratch_types=[
            pltpu.VMEM((n_chunk, CHUNK), jnp.int32),
            pltpu.VMEM((n_chunk, CHUNK), jnp.int32),
            pltpu.VMEM((b_per_w, DIM), jnp.float32),
            pltpu.VMEM((b_per_w, DIM), jnp.float32),
            pltpu.VMEM((b_per_w,), jnp.float32),
            pltpu.SemaphoreType.DMA((n_chunk,)),
        ],
        compiler_params=pltpu.CompilerParams(
            needs_layout_passes=False,
            use_tc_tiling_on_sc=False,
        ),
    )
    def k(users_hbm, items_hbm, ut_hbm, it_hbm, out_hbm,
          uidx_v, iidx_v, urows_v, irows_v, out_v, sems):
        wid = lax.axis_index("s") * NUM_CORES + lax.axis_index("c")
        base = wid * b_per_w
        lane = lax.iota(jnp.int32, LANES)

        for j in range(n_chunk):
            pltpu.sync_copy(users_hbm.at[pl.ds(base + j * CHUNK, CHUNK)],
                            uidx_v.at[j])
            pltpu.sync_copy(items_hbm.at[pl.ds(base + j * CHUNK, CHUNK)],
                            iidx_v.at[j])

        copies = []
        for j in range(n_chunk):
            cu = pltpu.async_copy(ut_hbm.at[uidx_v.at[j]],
                                  urows_v.at[pl.ds(j * CHUNK, CHUNK)],
                                  sems.at[j])
            ci = pltpu.async_copy(it_hbm.at[iidx_v.at[j]],
                                  irows_v.at[pl.ds(j * CHUNK, CHUNK)],
                                  sems.at[j])
            copies.append((cu, ci))

        for j in range(n_chunk):
            cu, ci = copies[j]
            cu.wait()
            ci.wait()

            def group_body(g, _, j=j):
                row0 = (j * groups_per_chunk) * LANES + g * LANES
                row_idx = row0 + lane
                accs = [jnp.zeros((LANES,), jnp.float32) for _ in range(4)]
                for d in range(DIM):
                    col = jnp.full((LANES,), d, jnp.int32)
                    u = plsc.load_gather(urows_v, [row_idx, col])
                    v = plsc.load_gather(irows_v, [row_idx, col])
                    accs[d % 4] = accs[d % 4] + u * v
                acc = (accs[0] + accs[1]) + (accs[2] + accs[3])
                out_v[pl.ds(row0, LANES)] = acc
                return 0

            lax.fori_loop(0, groups_per_chunk, group_body, 0)

        pltpu.sync_copy(out_v, out_hbm.at[pl.ds(base, b_per_w)])

    return k(users, items, user_table, item_table)


def kernel(users, items, user_table, item_table):
    users = users.astype(jnp.int32)
    items = items.astype(jnp.int32)
    return _scores_sc(users, items, user_table, item_table)


# speedup vs baseline: 1.0014x; 1.0014x over previous
import functools

import jax
import jax.numpy as jnp
from jax import lax
from jax.experimental import pallas as pl
from jax.experimental.pallas import tpu as pltpu
from jax.experimental.pallas import tpu_sc as plsc

DIM = 64
LANES = 16
NUM_CORES = 2
NUM_SUBCORES = 16
NUM_WORKERS = NUM_CORES * NUM_SUBCORES
CHUNK = 128
D_UNROLL = 8


def _scores_sc(users, items, user_table, item_table):
    batch = users.shape[0]
    b_per_w = batch // NUM_WORKERS
    n_chunk = b_per_w // CHUNK
    groups_per_chunk = CHUNK // LANES

    mesh = plsc.VectorSubcoreMesh(core_axis_name="c", subcore_axis_name="s")

    @functools.partial(
        pl.kernel,
        mesh=mesh,
        out_type=jax.ShapeDtypeStruct((batch,), jnp.float32),
        scratch_types=[
            pltpu.VMEM((n_chunk, CHUNK), jnp.int32),
            pltpu.VMEM((n_chunk, CHUNK), jnp.int32),
            pltpu.VMEM((b_per_w, DIM), jnp.float32),
            pltpu.VMEM((b_per_w, DIM), jnp.float32),
            pltpu.VMEM((b_per_w,), jnp.float32),
            pltpu.SemaphoreType.DMA((n_chunk,)),
        ],
        compiler_params=pltpu.CompilerParams(
            needs_layout_passes=False,
            use_tc_tiling_on_sc=False,
        ),
    )
    def k(users_hbm, items_hbm, ut_hbm, it_hbm, out_hbm,
          uidx_v, iidx_v, urows_v, irows_v, out_v, sems):
        wid = lax.axis_index("s") * NUM_CORES + lax.axis_index("c")
        base = wid * b_per_w
        lane = lax.iota(jnp.int32, LANES)

        for j in range(n_chunk):
            pltpu.sync_copy(users_hbm.at[pl.ds(base + j * CHUNK, CHUNK)],
                            uidx_v.at[j])
            pltpu.sync_copy(items_hbm.at[pl.ds(base + j * CHUNK, CHUNK)],
                            iidx_v.at[j])

        copies = []
        for j in range(n_chunk):
            cu = pltpu.async_copy(ut_hbm.at[uidx_v.at[j]],
                                  urows_v.at[pl.ds(j * CHUNK, CHUNK)],
                                  sems.at[j])
            ci = pltpu.async_copy(it_hbm.at[iidx_v.at[j]],
                                  irows_v.at[pl.ds(j * CHUNK, CHUNK)],
                                  sems.at[j])
            copies.append((cu, ci))

        for j in range(n_chunk):
            cu, ci = copies[j]
            cu.wait()
            ci.wait()

            def group_body(g, _, j=j):
                row0 = (j * groups_per_chunk) * LANES + g * LANES
                row_idx = row0 + lane
                zero = jnp.zeros((LANES,), jnp.float32)

                def d_body(db, accs, row_idx=row_idx):
                    a0, a1 = accs
                    d0 = db * D_UNROLL
                    for t in range(D_UNROLL):
                        col = jnp.full((LANES,), d0 + t, jnp.int32)
                        u = plsc.load_gather(urows_v, [row_idx, col])
                        v = plsc.load_gather(irows_v, [row_idx, col])
                        if t % 2 == 0:
                            a0 = a0 + u * v
                        else:
                            a1 = a1 + u * v
                    return (a0, a1)

                a0, a1 = lax.fori_loop(0, DIM // D_UNROLL, d_body, (zero, zero))
                out_v[pl.ds(row0, LANES)] = a0 + a1
                return 0

            lax.fori_loop(0, groups_per_chunk, group_body, 0)

        pltpu.sync_copy(out_v, out_hbm.at[pl.ds(base, b_per_w)])

    return k(users, items, user_table, item_table)


def kernel(users, items, user_table, item_table):
    users = users.astype(jnp.int32)
    items = items.astype(jnp.int32)
    return _scores_sc(users, items, user_table, item_table)
